# baseline (speedup 1.0000x reference)
import functools

import jax
import jax.numpy as jnp
from jax import lax
from jax.experimental import pallas as pl
from jax.experimental.pallas import tpu as pltpu, tpu_sc as plsc

_ALPHA = 0.01
_L = 16


def _loss_kernel_body(T, E, K, NW, idx_hbm, probs_hbm, out_hbm,
                      idx_v, probs_v, hist_v, part_v, shared, allv, out_v):
    rows_w = T // NW
    n_idx = rows_w * K
    n_probs = rows_w * E
    wid = lax.axis_index("s")
    lanes = lax.iota(jnp.int32, _L)
    ones = jnp.full((_L,), 1.0, dtype=jnp.float32)
    zeros = jnp.zeros((_L,), dtype=jnp.float32)

    pltpu.sync_copy(idx_hbm.at[pl.ds(wid * n_idx, n_idx)], idx_v)
    pltpu.sync_copy(probs_hbm.at[pl.ds(wid * n_probs, n_probs)], probs_v)

    def zero_step(i, _):
        hist_v[pl.ds(i * _L, _L)] = zeros
        return 0
    lax.fori_loop(0, _L * E // _L, zero_step, 0)

    lane_base = lanes * E

    def hist_step(i, _):
        v = idx_v[pl.ds(i * _L, _L)]
        plsc.addupdate_scatter(hist_v, [lane_base + v], ones)
        return 0
    lax.fori_loop(0, n_idx // _L, hist_step, 0)

    g = E // _L

    def psum_step(i, acc):
        base = i * E
        return tuple(acc[j] + probs_v[pl.ds(base + j * _L, _L)]
                     for j in range(g))
    acc = lax.fori_loop(0, rows_w, psum_step,
                        tuple(zeros for _ in range(g)))

    for j in range(g):
        cnt = zeros
        for r in range(_L):
            cnt = cnt + hist_v[pl.ds(r * E + j * _L, _L)]
        part_v[pl.ds(j * _L, _L)] = cnt
        part_v[pl.ds(E + j * _L, _L)] = acc[j]

    pltpu.sync_copy(part_v, shared.at[wid])
    plsc.subcore_barrier()

    @pl.when(wid == 0)
    def _():
        pltpu.sync_copy(shared, allv)
        tot = []
        for j in range(2 * g):
            t = zeros
            for w in range(NW):
                t = t + allv[w, pl.ds(j * _L, _L)]
            tot.append(t)
        dot = zeros
        for j in range(g):
            dot = dot + tot[j] * tot[g + j]
        scale = _ALPHA * E / (float(T) * float(T))
        s = jnp.sum(dot, axis=0) * scale
        out_v[...] = jnp.full((_L,), 1.0, jnp.float32) * s
        pltpu.sync_copy(out_v, out_hbm)


def kernel(indices, weights, probs, n_experts):
    del weights, n_experts
    T, K = indices.shape
    E = probs.shape[-1]
    NW = 16
    idx_flat = indices.astype(jnp.int32).reshape(-1)
    probs_flat = probs.reshape(-1)
    rows_w = T // NW

    mesh = plsc.VectorSubcoreMesh(core_axis_name="c", subcore_axis_name="s",
                                  num_cores=1)
    body = functools.partial(_loss_kernel_body, T, E, K, NW)
    out = pl.kernel(
        body,
        out_type=jax.ShapeDtypeStruct((_L,), jnp.float32),
        mesh=mesh,
        compiler_params=pltpu.CompilerParams(needs_layout_passes=False),
        scratch_types=[
            pltpu.VMEM((rows_w * K,), jnp.int32),
            pltpu.VMEM((rows_w * E,), jnp.float32),
            pltpu.VMEM((_L * E,), jnp.float32),
            pltpu.VMEM((2 * E,), jnp.float32),
            pltpu.VMEM_SHARED((NW, 2 * E), jnp.float32),
            pltpu.VMEM((NW, 2 * E), jnp.float32),
            pltpu.VMEM((_L,), jnp.float32),
        ],
    )(idx_flat, probs_flat)
    return out[0]


# speedup vs baseline: 1.0860x; 1.0860x over previous
import functools

import jax
import jax.numpy as jnp
from jax import lax
from jax.experimental import pallas as pl
from jax.experimental.pallas import tpu as pltpu, tpu_sc as plsc

_ALPHA = 0.01
_L = 16


def _loss_kernel_body(T, E, K, NW, idx_hbm, probs_hbm, out_hbm,
                      idx_v, probs_v, hist_v, part_v, shared, allv, out_v,
                      sem):
    rows_w = T // NW
    n_idx = rows_w * K
    n_probs = rows_w * E
    wid = lax.axis_index("s")
    lanes = lax.iota(jnp.int32, _L)
    ones = jnp.full((_L,), 1.0, dtype=jnp.float32)
    zeros = jnp.zeros((_L,), dtype=jnp.float32)

    probs_cp = pltpu.make_async_copy(
        probs_hbm.at[pl.ds(wid * n_probs, n_probs)], probs_v, sem)
    probs_cp.start()
    pltpu.sync_copy(idx_hbm.at[pl.ds(wid * n_idx, n_idx)], idx_v)

    for i in range(E):
        hist_v[pl.ds(i * _L, _L)] = zeros

    lane_base = lanes * E
    n_vec = n_idx // _L
    UH = 16

    def hist_step(i, _):
        base = i * UH
        for u in range(UH):
            v = idx_v[pl.ds((base + u) * _L, _L)]
            plsc.addupdate_scatter(hist_v, [lane_base + v], ones)
        return 0
    lax.fori_loop(0, n_vec // UH, hist_step, 0)

    probs_cp.wait()

    g = E // _L
    UR = 8

    def psum_step(i, acc):
        acc = list(acc)
        base = i * UR * E
        for u in range(UR):
            for j in range(g):
                acc[j] = acc[j] + probs_v[pl.ds(base + u * E + j * _L, _L)]
        return tuple(acc)
    acc = lax.fori_loop(0, rows_w // UR, psum_step,
                        tuple(zeros for _ in range(g)))

    for j in range(g):
        cnt = zeros
        for r in range(_L):
            cnt = cnt + hist_v[pl.ds(r * E + j * _L, _L)]
        part_v[pl.ds(j * _L, _L)] = cnt
        part_v[pl.ds(E + j * _L, _L)] = acc[j]

    pltpu.sync_copy(part_v, shared.at[wid])
    plsc.subcore_barrier()

    @pl.when(wid == 0)
    def _():
        pltpu.sync_copy(shared, allv)
        tot = []
        for j in range(2 * g):
            t = zeros
            for w in range(NW):
                t = t + allv[w, pl.ds(j * _L, _L)]
            tot.append(t)
        dot = zeros
        for j in range(g):
            dot = dot + tot[j] * tot[g + j]
        scale = _ALPHA * E / (float(T) * float(T))
        s = jnp.sum(dot, axis=0) * scale
        out_v[...] = jnp.full((_L,), 1.0, jnp.float32) * s
        pltpu.sync_copy(out_v, out_hbm)


def kernel(indices, weights, probs, n_experts):
    del weights, n_experts
    T, K = indices.shape
    E = probs.shape[-1]
    NW = 16
    idx_flat = indices.astype(jnp.int32).reshape(-1)
    probs_flat = probs.reshape(-1)
    rows_w = T // NW

    mesh = plsc.VectorSubcoreMesh(core_axis_name="c", subcore_axis_name="s",
                                  num_cores=1)
    body = functools.partial(_loss_kernel_body, T, E, K, NW)
    out = pl.kernel(
        body,
        out_type=jax.ShapeDtypeStruct((_L,), jnp.float32),
        mesh=mesh,
        compiler_params=pltpu.CompilerParams(needs_layout_passes=False),
        scratch_types=[
            pltpu.VMEM((rows_w * K,), jnp.int32),
            pltpu.VMEM((rows_w * E,), jnp.float32),
            pltpu.VMEM((_L * E,), jnp.float32),
            pltpu.VMEM((2 * E,), jnp.float32),
            pltpu.VMEM_SHARED((NW, 2 * E), jnp.float32),
            pltpu.VMEM((NW, 2 * E), jnp.float32),
            pltpu.VMEM((_L,), jnp.float32),
            pltpu.SemaphoreType.DMA,
        ],
    )(idx_flat, probs_flat)
    return out[0]


# speedup vs baseline: 1.8600x; 1.7127x over previous
import functools

import jax
import jax.numpy as jnp
from jax import lax
from jax.experimental import pallas as pl
from jax.experimental.pallas import tpu as pltpu

_ALPHA = 0.01


def _body(nblk, T, E, scale, idx_ref, probs_ref, out_ref, acc_cnt, acc_p):
    i = pl.program_id(0)

    @pl.when(i == 0)
    def _():
        acc_cnt[...] = jnp.zeros_like(acc_cnt)
        acc_p[...] = jnp.zeros_like(acc_p)

    idxb = idx_ref[...]
    e_iota = lax.broadcasted_iota(jnp.int32, (E, 1, 1), 0)
    cmp = (idxb[None, :, :] == e_iota).astype(jnp.float32)
    acc_cnt[...] += jnp.sum(cmp, axis=1)

    pb = probs_ref[...]
    pb3 = pb.reshape(pb.shape[0] // 8, 8, E)
    acc_p[...] += jnp.sum(pb3, axis=0)

    @pl.when(i == nblk - 1)
    def _():
        cnt = jnp.sum(acc_cnt[...], axis=1)
        p = jnp.sum(acc_p[...], axis=0)
        out_ref[...] = (jnp.sum(cnt * p) * scale).reshape(1, 1)


def kernel(indices, weights, probs, n_experts):
    del weights, n_experts
    T, K = indices.shape
    E = probs.shape[-1]
    nblk = 8
    lanes = 256
    n_flat = T * K
    rows = n_flat // lanes
    idx2d = indices.astype(jnp.int32).reshape(rows, lanes)
    rows_b = rows // nblk
    rows_p = T // nblk
    scale = _ALPHA * E / (float(T) * float(T))

    out = pl.pallas_call(
        functools.partial(_body, nblk, T, E, scale),
        grid=(nblk,),
        in_specs=[
            pl.BlockSpec((rows_b, lanes), lambda i: (i, 0)),
            pl.BlockSpec((rows_p, E), lambda i: (i, 0)),
        ],
        out_specs=pl.BlockSpec((1, 1), lambda i: (0, 0)),
        out_shape=jax.ShapeDtypeStruct((1, 1), jnp.float32),
        scratch_shapes=[
            pltpu.VMEM((E, lanes), jnp.float32),
            pltpu.VMEM((8, E), jnp.float32),
        ],
        compiler_params=pltpu.CompilerParams(
            dimension_semantics=("arbitrary",),
        ),
    )(idx2d, probs)
    return out[0, 0]


# speedup vs baseline: 2.0427x; 1.0982x over previous
import functools

import jax
import jax.numpy as jnp
from jax import lax
from jax.experimental import pallas as pl
from jax.experimental.pallas import tpu as pltpu

_ALPHA = 0.01


def _body(nblk, scale, idx_ref, probs_ref, out_ref, acc8, acc_p):
    i = pl.program_id(0)

    @pl.when(i == 0)
    def _():
        acc8[...] = jnp.zeros_like(acc8)
        acc_p[...] = jnp.zeros_like(acc_p)

    idxb = idx_ref[...]
    iota8 = lax.broadcasted_iota(jnp.int32, (8, 1), 0)
    c8 = jnp.zeros((8, 8), jnp.float32)
    for r in range(8):
        strip = idxb[r:r + 1, :]
        hm = ((strip >> 3) == iota8).astype(jnp.float32)
        lm = ((strip & 7) == iota8).astype(jnp.float32)
        c8 += lax.dot_general(
            hm, lm, (((1,), (1,)), ((), ())),
            preferred_element_type=jnp.float32)
    acc8[...] += c8

    pb = probs_ref[...]
    ones = jnp.ones((1, pb.shape[0]), jnp.float32)
    acc_p[...] += lax.dot_general(
        ones, pb, (((1,), (0,)), ((), ())),
        preferred_element_type=jnp.float32)

    @pl.when(i == nblk - 1)
    def _():
        e_row = lax.broadcasted_iota(jnp.int32, (8, 64), 1)
        h_row = lax.broadcasted_iota(jnp.int32, (8, 64), 0)
        hmask = ((e_row >> 3) == h_row).astype(jnp.float32)
        e_col = lax.broadcasted_iota(jnp.int32, (64, 8), 0)
        l_col = lax.broadcasted_iota(jnp.int32, (64, 8), 1)
        lmask = ((e_col & 7) == l_col).astype(jnp.float32)
        psum8 = lax.dot_general(
            hmask * acc_p[...], lmask, (((1,), (0,)), ((), ())),
            preferred_element_type=jnp.float32)
        out_ref[...] = jnp.sum(acc8[...] * psum8, keepdims=True) * scale


def kernel(indices, weights, probs, n_experts):
    del weights, n_experts
    T, K = indices.shape
    E = probs.shape[-1]
    nblk = 8
    n_flat = T * K
    lb = n_flat // (nblk * 8)
    idx2d = indices.astype(jnp.int32).reshape(nblk * 8, lb)
    rows_p = T // nblk
    scale = _ALPHA * E / (float(T) * float(T))

    out = pl.pallas_call(
        functools.partial(_body, nblk, scale),
        grid=(nblk,),
        in_specs=[
            pl.BlockSpec((8, lb), lambda i: (i, 0)),
            pl.BlockSpec((rows_p, E), lambda i: (i, 0)),
        ],
        out_specs=pl.BlockSpec((1, 1), lambda i: (0, 0)),
        out_shape=jax.ShapeDtypeStruct((1, 1), jnp.float32),
        scratch_shapes=[
            pltpu.VMEM((8, 8), jnp.float32),
            pltpu.VMEM((1, E), jnp.float32),
        ],
        compiler_params=pltpu.CompilerParams(
            dimension_semantics=("arbitrary",),
        ),
    )(idx2d, probs)
    return out[0, 0]
